# baseline (speedup 1.0000x reference)
import functools

import jax
import jax.numpy as jnp
from jax import lax
from jax.experimental import pallas as pl
from jax.experimental.pallas import tpu as pltpu
from jax.experimental.pallas import tpu_sc as plsc

_N = 100000
_C = 100
_NW = 32
_ROWS_W = 3136
_CHUNK = 448
_CW = _CHUNK * _C
_GROUPS = _CHUNK // 16
_TAIL = 96
_TAIL_GROUPS = _TAIL // 16

_mesh = plsc.VectorSubcoreMesh(core_axis_name="c", subcore_axis_name="s")


@functools.partial(
    pl.kernel,
    out_type=jax.ShapeDtypeStruct((_N * _C,), jnp.float32),
    mesh=_mesh,
    scratch_types=[
        pltpu.VMEM((_ROWS_W,), jnp.int32),
        pltpu.VMEM((128,), jnp.int32),
        pltpu.VMEM((_CW,), jnp.float32),
        pltpu.VMEM((_CW,), jnp.float32),
        pltpu.VMEM((_CHUNK,), jnp.int32),
        pltpu.VMEM((_CHUNK,), jnp.int32),
        pltpu.SemaphoreType.DMA,
        pltpu.SemaphoreType.DMA,
    ],
    compiler_params=pltpu.CompilerParams(needs_layout_passes=False),
)
def _onehot_sc(zp_hbm, tab_hbm, zero_hbm, out_hbm,
               zbuf, tabv, buf0, buf1, sv0, sv1, sem0, sem1):
    wid = lax.axis_index("s") * 2 + lax.axis_index("c")
    base = wid * _ROWS_W

    pltpu.sync_copy(zp_hbm.at[pl.ds(base, _ROWS_W)], zbuf)
    pltpu.sync_copy(tab_hbm, tabv)
    cp0 = pltpu.async_copy(zero_hbm, buf0, sem0)
    cp1 = pltpu.async_copy(zero_hbm, buf1, sem1)
    cp0.wait()
    cp1.wait()

    lane100 = lax.broadcasted_iota(jnp.int32, (16,), 0) * 100
    ones = jnp.ones((16,), jnp.float32)
    zeros = jnp.zeros((16,), jnp.float32)

    bufs = (buf0, buf1)
    saves = (sv0, sv1)
    sems = (sem0, sem1)

    def out_dst(c, nrows=_CHUNK):
        return out_hbm.at[pl.ds((base + c * _CHUNK) * _C, nrows * _C)]

    def set_chunk(c, buf, sv, ngroups):
        for g in range(ngroups):
            z = zbuf[pl.ds(c * _CHUNK + 16 * g, 16)]
            idx = plsc.load_gather(tabv, [z])
            offs = lane100 + (idx + (1600 * g))
            plsc.store_scatter(buf, [offs], ones)
            sv[pl.ds(16 * g, 16)] = offs

    def clear_chunk(buf, sv):
        for g in range(_GROUPS):
            offs = sv[pl.ds(16 * g, 16)]
            plsc.store_scatter(buf, [offs], zeros)

    for c in range(6):
        b = c % 2
        if c >= 2:
            pltpu.make_async_copy(bufs[b], out_dst(c - 2), sems[b]).wait()
            clear_chunk(bufs[b], saves[b])
        set_chunk(c, bufs[b], saves[b], _GROUPS)
        pltpu.async_copy(bufs[b], out_dst(c), sems[b])

    is_last = wid == (_NW - 1)

    @pl.when(jnp.logical_not(is_last))
    def _():
        pltpu.make_async_copy(bufs[0], out_dst(4), sems[0]).wait()
        clear_chunk(bufs[0], saves[0])
        set_chunk(6, bufs[0], saves[0], _GROUPS)
        pltpu.async_copy(bufs[0], out_dst(6), sems[0])
        pltpu.make_async_copy(bufs[1], out_dst(5), sems[1]).wait()
        pltpu.make_async_copy(bufs[0], out_dst(6), sems[0]).wait()

    @pl.when(is_last)
    def _():
        pltpu.make_async_copy(bufs[0], out_dst(4), sems[0]).wait()
        clear_chunk(bufs[0], saves[0])
        set_chunk(6, bufs[0], saves[0], _TAIL_GROUPS)
        pltpu.async_copy(bufs[0].at[pl.ds(0, _TAIL * _C)], out_dst(6, _TAIL), sems[0])
        pltpu.make_async_copy(bufs[1], out_dst(5), sems[1]).wait()
        pltpu.make_async_copy(bufs[0].at[pl.ds(0, _TAIL * _C)], out_dst(6, _TAIL), sems[0]).wait()


def kernel(Z, z_to_index):
    zp = jnp.pad(Z, (0, _NW * _ROWS_W - _N))
    tab = jnp.pad(z_to_index, (0, 128 - z_to_index.shape[0]))
    zero = jnp.zeros((_CW,), jnp.float32)
    return _onehot_sc(zp, tab, zero).reshape(_N, _C)


# speedup vs baseline: 2.2685x; 2.2685x over previous
import functools

import jax
import jax.numpy as jnp
from jax import lax
from jax.experimental import pallas as pl
from jax.experimental.pallas import tpu as pltpu
from jax.experimental.pallas import tpu_sc as plsc

_N = 100000
_C = 100
_NW = 32
_ROWS_W = 3136
_CHUNK = 448
_GROUPS = _CHUNK // 16
_TAIL = 96
_TAIL_GROUPS = _TAIL // 16

_mesh = plsc.VectorSubcoreMesh(core_axis_name="c", subcore_axis_name="s")


@functools.partial(
    pl.kernel,
    out_type=jax.ShapeDtypeStruct((_N, _C), jnp.float32),
    mesh=_mesh,
    scratch_types=[
        pltpu.VMEM((_ROWS_W,), jnp.int32),
        pltpu.VMEM((128,), jnp.int32),
        pltpu.VMEM((_CHUNK, _C), jnp.float32),
        pltpu.VMEM((_CHUNK, _C), jnp.float32),
        pltpu.VMEM((_CHUNK,), jnp.int32),
        pltpu.VMEM((_CHUNK,), jnp.int32),
        pltpu.SemaphoreType.DMA,
        pltpu.SemaphoreType.DMA,
    ],
    compiler_params=pltpu.CompilerParams(needs_layout_passes=False),
)
def _onehot_sc(zp_hbm, tab_hbm, out_hbm,
               zbuf, tabv, buf0, buf1, sv0, sv1, sem0, sem1):
    wid = lax.axis_index("s") * 2 + lax.axis_index("c")
    base = wid * _ROWS_W

    pltpu.sync_copy(zp_hbm.at[pl.ds(base, _ROWS_W)], zbuf)
    pltpu.sync_copy(tab_hbm, tabv)

    lanes = lax.broadcasted_iota(jnp.int32, (16,), 0)
    ones = jnp.ones((16,), jnp.float32)
    zeros = jnp.zeros((16,), jnp.float32)

    def _zero_row(r, _):
        for buf in (buf0, buf1):
            for g in range(6):
                buf[r, pl.ds(16 * g, 16)] = zeros
            buf[r, pl.ds(_C - 16, 16)] = zeros
        return 0

    lax.fori_loop(0, _CHUNK, _zero_row, 0)

    bufs = (buf0, buf1)
    saves = (sv0, sv1)
    sems = (sem0, sem1)

    def out_dst(c, nrows=_CHUNK):
        return out_hbm.at[pl.ds(base + c * _CHUNK, nrows), :]

    def set_chunk(c, buf, sv, ngroups):
        for g in range(ngroups):
            z = zbuf[pl.ds(c * _CHUNK + 16 * g, 16)]
            idx = plsc.load_gather(tabv, [z])
            rows = lanes + (16 * g)
            plsc.store_scatter(buf, [rows, idx], ones)
            sv[pl.ds(16 * g, 16)] = idx

    def clear_chunk(buf, sv):
        for g in range(_GROUPS):
            idx = sv[pl.ds(16 * g, 16)]
            rows = lanes + (16 * g)
            plsc.store_scatter(buf, [rows, idx], zeros)

    for c in range(6):
        b = c % 2
        if c >= 2:
            pltpu.make_async_copy(bufs[b], out_dst(c - 2), sems[b]).wait()
            clear_chunk(bufs[b], saves[b])
        set_chunk(c, bufs[b], saves[b], _GROUPS)
        pltpu.async_copy(bufs[b], out_dst(c), sems[b])

    is_last = wid == (_NW - 1)

    @pl.when(jnp.logical_not(is_last))
    def _():
        pltpu.make_async_copy(bufs[0], out_dst(4), sems[0]).wait()
        clear_chunk(bufs[0], saves[0])
        set_chunk(6, bufs[0], saves[0], _GROUPS)
        pltpu.async_copy(bufs[0], out_dst(6), sems[0])
        pltpu.make_async_copy(bufs[1], out_dst(5), sems[1]).wait()
        pltpu.make_async_copy(bufs[0], out_dst(6), sems[0]).wait()

    @pl.when(is_last)
    def _():
        pltpu.make_async_copy(bufs[0], out_dst(4), sems[0]).wait()
        clear_chunk(bufs[0], saves[0])
        set_chunk(6, bufs[0], saves[0], _TAIL_GROUPS)
        pltpu.async_copy(bufs[0].at[pl.ds(0, _TAIL), :], out_dst(6, _TAIL), sems[0])
        pltpu.make_async_copy(bufs[1], out_dst(5), sems[1]).wait()
        pltpu.make_async_copy(bufs[0].at[pl.ds(0, _TAIL), :], out_dst(6, _TAIL), sems[0]).wait()


def kernel(Z, z_to_index):
    zp = jnp.pad(Z, (0, _NW * _ROWS_W - _N))
    tab = jnp.pad(z_to_index, (0, 128 - z_to_index.shape[0]))
    return _onehot_sc(zp, tab)


# speedup vs baseline: 3.2454x; 1.4306x over previous
import functools

import jax
import jax.numpy as jnp
from jax import lax
from jax.experimental import pallas as pl
from jax.experimental.pallas import tpu as pltpu
from jax.experimental.pallas import tpu_sc as plsc

_N = 100000
_C = 100
_NW = 32
_CB = 384
_GROUPS = _CB // 16
_COLS_HEAVY = 9 * _CB
_COLS_LIGHT = 8 * _CB
_TAIL = 256
_TAIL_GROUPS = _TAIL // 16
_TAIL_BASE = 260 * _CB
_TAIL_USED = _N - _TAIL_BASE
_ZPAD = 100352
_TABOFF = _ZPAD

_mesh = plsc.VectorSubcoreMesh(core_axis_name="c", subcore_axis_name="s")


@functools.partial(
    pl.kernel,
    out_type=(
        jax.ShapeDtypeStruct((_C, _N), jnp.float32),
        jax.ShapeDtypeStruct((_C, _TAIL), jnp.float32),
    ),
    mesh=_mesh,
    scratch_types=[
        pltpu.VMEM((_COLS_HEAVY,), jnp.int32),
        pltpu.VMEM((128,), jnp.int32),
        pltpu.VMEM((_C, _CB), jnp.float32),
        pltpu.VMEM((_C, _CB), jnp.float32),
        pltpu.VMEM((_C, _TAIL), jnp.float32),
        pltpu.VMEM((_CB,), jnp.int32),
        pltpu.VMEM((_CB,), jnp.int32),
        pltpu.SemaphoreType.DMA,
        pltpu.SemaphoreType.DMA,
        pltpu.SemaphoreType.DMA,
    ],
    compiler_params=pltpu.CompilerParams(needs_layout_passes=False),
)
def _onehot_sc(zt_hbm, out_hbm, tail_hbm,
               zbuf, tabv, buf0, buf1, tbuf, sv0, sv1, sem0, sem1, sem2):
    wid = lax.axis_index("s") * 2 + lax.axis_index("c")
    base = jnp.minimum(wid, 4) * _COLS_HEAVY + jnp.maximum(wid - 4, 0) * _COLS_LIGHT

    pltpu.sync_copy(zt_hbm.at[pl.ds(base, _COLS_HEAVY)], zbuf)
    pltpu.sync_copy(zt_hbm.at[pl.ds(_TABOFF, 128)], tabv)

    lanes = lax.broadcasted_iota(jnp.int32, (16,), 0)
    ones = jnp.ones((16,), jnp.float32)
    zeros = jnp.zeros((16,), jnp.float32)

    def zero_buf(buf, ngroups):
        def row(r, _):
            def col(g, _):
                buf[r, pl.ds(16 * g, 16)] = zeros
                return 0
            return lax.fori_loop(0, ngroups, col, 0)
        lax.fori_loop(0, _C, row, 0)

    def out_dst(c):
        return out_hbm.at[:, pl.ds(base + c * _CB, _CB)]

    def set_chunk(c, buf, sv, ngroups):
        def g_body(g, _):
            z = zbuf[pl.ds(c * _CB + 16 * g, 16)]
            idx = plsc.load_gather(tabv, [z])
            cols = lanes + 16 * g
            plsc.store_scatter(buf, [idx, cols], ones)
            if sv is not None:
                sv[pl.ds(16 * g, 16)] = idx
            return 0
        lax.fori_loop(0, ngroups, g_body, 0)

    def clear_chunk(buf, sv):
        def g_body(g, _):
            idx = sv[pl.ds(16 * g, 16)]
            cols = lanes + 16 * g
            plsc.store_scatter(buf, [idx, cols], zeros)
            return 0
        lax.fori_loop(0, _GROUPS, g_body, 0)

    zero_buf(buf0, _GROUPS)
    set_chunk(0, buf0, sv0, _GROUPS)
    pltpu.async_copy(buf0, out_dst(0), sem0)
    zero_buf(buf1, _GROUPS)
    set_chunk(1, buf1, sv1, _GROUPS)
    pltpu.async_copy(buf1, out_dst(1), sem1)

    def pair(k, _):
        c0 = 2 * k
        pltpu.make_async_copy(buf0, out_dst(c0 - 2), sem0).wait()
        clear_chunk(buf0, sv0)
        set_chunk(c0, buf0, sv0, _GROUPS)
        pltpu.async_copy(buf0, out_dst(c0), sem0)
        c1 = c0 + 1
        pltpu.make_async_copy(buf1, out_dst(c1 - 2), sem1).wait()
        clear_chunk(buf1, sv1)
        set_chunk(c1, buf1, sv1, _GROUPS)
        pltpu.async_copy(buf1, out_dst(c1), sem1)
        return 0

    lax.fori_loop(1, 4, pair, 0)

    @pl.when(wid < 4)
    def _():
        pltpu.make_async_copy(buf0, out_dst(6), sem0).wait()
        clear_chunk(buf0, sv0)
        set_chunk(8, buf0, sv0, _GROUPS)
        pltpu.async_copy(buf0, out_dst(8), sem0)
        pltpu.make_async_copy(buf1, out_dst(7), sem1).wait()
        pltpu.make_async_copy(buf0, out_dst(8), sem0).wait()

    @pl.when(wid == _NW - 1)
    def _():
        zero_buf(tbuf, _TAIL_GROUPS)
        set_chunk(8, tbuf, None, _TAIL_GROUPS)
        pltpu.async_copy(tbuf, tail_hbm, sem2)
        pltpu.make_async_copy(buf0, out_dst(6), sem0).wait()
        pltpu.make_async_copy(buf1, out_dst(7), sem1).wait()
        pltpu.make_async_copy(tbuf, tail_hbm, sem2).wait()

    @pl.when(jnp.logical_and(wid >= 4, wid < _NW - 1))
    def _():
        pltpu.make_async_copy(buf0, out_dst(6), sem0).wait()
        pltpu.make_async_copy(buf1, out_dst(7), sem1).wait()


def kernel(Z, z_to_index):
    zt = jnp.concatenate([
        Z,
        jnp.zeros((_ZPAD - _N,), jnp.int32),
        z_to_index,
        jnp.zeros((128 - z_to_index.shape[0],), jnp.int32),
    ])
    main, tail = _onehot_sc(zt)
    out = main.T
    upd = tail.T[:_TAIL_USED]
    return lax.dynamic_update_slice(out, upd, (_TAIL_BASE, 0))


# speedup vs baseline: 4.3855x; 1.3513x over previous
import functools

import jax
import jax.numpy as jnp
from jax import lax
from jax.experimental import pallas as pl
from jax.experimental.pallas import tpu as pltpu
from jax.experimental.pallas import tpu_sc as plsc

_N = 100000
_C = 100
_NW = 32
_CB = 384
_GROUPS = _CB // 16
_COLS_HEAVY = 9 * _CB
_COLS_LIGHT = 8 * _CB
_TAIL = 256
_TAIL_GROUPS = _TAIL // 16
_TAIL_BASE = 260 * _CB
_TAIL_USED = _N - _TAIL_BASE
_ZPAD = 100352
_TABOFF = _ZPAD

_mesh = plsc.VectorSubcoreMesh(core_axis_name="c", subcore_axis_name="s")


@functools.partial(
    pl.kernel,
    out_type=(
        jax.ShapeDtypeStruct((_C, _N), jnp.float32),
        jax.ShapeDtypeStruct((_C, _TAIL), jnp.float32),
    ),
    mesh=_mesh,
    scratch_types=[
        pltpu.VMEM((_COLS_HEAVY,), jnp.int32),
        pltpu.VMEM((128,), jnp.int32),
        pltpu.VMEM((_C, _CB), jnp.float32),
        pltpu.VMEM((_C, _CB), jnp.float32),
        pltpu.VMEM((_C, _TAIL), jnp.float32),
        pltpu.VMEM((_CB,), jnp.int32),
        pltpu.VMEM((_CB,), jnp.int32),
        pltpu.SemaphoreType.DMA,
        pltpu.SemaphoreType.DMA,
        pltpu.SemaphoreType.DMA,
    ],
    compiler_params=pltpu.CompilerParams(needs_layout_passes=False),
)
def _onehot_sc(zt_hbm, out_hbm, tail_hbm,
               zbuf, tabv, buf0, buf1, tbuf, sv0, sv1, sem0, sem1, sem2):
    wid = lax.axis_index("s") * 2 + lax.axis_index("c")
    base = jnp.minimum(wid, 4) * _COLS_HEAVY + jnp.maximum(wid - 4, 0) * _COLS_LIGHT

    pltpu.sync_copy(zt_hbm.at[pl.ds(base, _COLS_HEAVY)], zbuf)
    pltpu.sync_copy(zt_hbm.at[pl.ds(_TABOFF, 128)], tabv)

    lanes = lax.broadcasted_iota(jnp.int32, (16,), 0)
    ones = jnp.ones((16,), jnp.float32)
    zeros = jnp.zeros((16,), jnp.float32)

    def zero_buf(buf, ngroups):
        def row(r, _):
            for g in range(ngroups):
                buf[r, pl.ds(16 * g, 16)] = zeros
            return 0
        lax.fori_loop(0, _C, row, 0)

    def out_dst(c):
        return out_hbm.at[:, pl.ds(base + c * _CB, _CB)]

    def set_chunk(c, buf, sv, ngroups):
        zoff = c * _CB
        for g in range(ngroups):
            z = zbuf[pl.ds(zoff + 16 * g, 16)]
            idx = plsc.load_gather(tabv, [z])
            cols = lanes + 16 * g
            plsc.store_scatter(buf, [idx, cols], ones)
            if sv is not None:
                sv[pl.ds(16 * g, 16)] = idx

    def clear_chunk(buf, sv):
        for g in range(_GROUPS):
            idx = sv[pl.ds(16 * g, 16)]
            cols = lanes + 16 * g
            plsc.store_scatter(buf, [idx, cols], zeros)

    zero_buf(buf0, _GROUPS)
    set_chunk(0, buf0, sv0, _GROUPS)
    pltpu.async_copy(buf0, out_dst(0), sem0)
    zero_buf(buf1, _GROUPS)
    set_chunk(1, buf1, sv1, _GROUPS)
    pltpu.async_copy(buf1, out_dst(1), sem1)

    def pair(k, _):
        c0 = 2 * k
        pltpu.make_async_copy(buf0, out_dst(c0 - 2), sem0).wait()
        clear_chunk(buf0, sv0)
        set_chunk(c0, buf0, sv0, _GROUPS)
        pltpu.async_copy(buf0, out_dst(c0), sem0)
        c1 = c0 + 1
        pltpu.make_async_copy(buf1, out_dst(c1 - 2), sem1).wait()
        clear_chunk(buf1, sv1)
        set_chunk(c1, buf1, sv1, _GROUPS)
        pltpu.async_copy(buf1, out_dst(c1), sem1)
        return 0

    lax.fori_loop(1, 4, pair, 0)

    @pl.when(wid < 4)
    def _():
        pltpu.make_async_copy(buf0, out_dst(6), sem0).wait()
        clear_chunk(buf0, sv0)
        set_chunk(8, buf0, sv0, _GROUPS)
        pltpu.async_copy(buf0, out_dst(8), sem0)
        pltpu.make_async_copy(buf1, out_dst(7), sem1).wait()
        pltpu.make_async_copy(buf0, out_dst(8), sem0).wait()

    @pl.when(wid == _NW - 1)
    def _():
        zero_buf(tbuf, _TAIL_GROUPS)
        set_chunk(8, tbuf, None, _TAIL_GROUPS)
        pltpu.async_copy(tbuf, tail_hbm, sem2)
        pltpu.make_async_copy(buf0, out_dst(6), sem0).wait()
        pltpu.make_async_copy(buf1, out_dst(7), sem1).wait()
        pltpu.make_async_copy(tbuf, tail_hbm, sem2).wait()

    @pl.when(jnp.logical_and(wid >= 4, wid < _NW - 1))
    def _():
        pltpu.make_async_copy(buf0, out_dst(6), sem0).wait()
        pltpu.make_async_copy(buf1, out_dst(7), sem1).wait()


def kernel(Z, z_to_index):
    zt = jnp.concatenate([
        Z,
        jnp.zeros((_ZPAD - _N,), jnp.int32),
        z_to_index,
        jnp.zeros((128 - z_to_index.shape[0],), jnp.int32),
    ])
    main, tail = _onehot_sc(zt)
    out = main.T
    upd = tail.T[:_TAIL_USED]
    return lax.dynamic_update_slice(out, upd, (_TAIL_BASE, 0))


# speedup vs baseline: 4.5600x; 1.0398x over previous
import functools

import jax
import jax.numpy as jnp
from jax import lax
from jax.experimental import pallas as pl
from jax.experimental.pallas import tpu as pltpu
from jax.experimental.pallas import tpu_sc as plsc

_N = 100000
_C = 100
_NW = 32
_CB = 384
_GROUPS = _CB // 16
_COLS_HEAVY = 9 * _CB
_COLS_LIGHT = 8 * _CB
_TAIL = 256
_TAIL_GROUPS = _TAIL // 16
_TAIL_BASE = 260 * _CB
_TAIL_USED = _N - _TAIL_BASE
_TAIL_SET_GROUPS = _TAIL_USED // 16

_mesh = plsc.VectorSubcoreMesh(core_axis_name="c", subcore_axis_name="s")


@functools.partial(
    pl.kernel,
    out_type=(
        jax.ShapeDtypeStruct((_C, _N), jnp.float32),
        jax.ShapeDtypeStruct((_C, _TAIL), jnp.float32),
    ),
    mesh=_mesh,
    scratch_types=[
        pltpu.VMEM((_COLS_LIGHT,), jnp.int32),
        pltpu.VMEM((_CB,), jnp.int32),
        pltpu.VMEM((_TAIL_USED,), jnp.int32),
        pltpu.VMEM((120,), jnp.int32),
        pltpu.VMEM((_C, _CB), jnp.float32),
        pltpu.VMEM((_C, _CB), jnp.float32),
        pltpu.VMEM((_C, _TAIL), jnp.float32),
        pltpu.VMEM((_CB,), jnp.int32),
        pltpu.VMEM((_CB,), jnp.int32),
        pltpu.SemaphoreType.DMA,
        pltpu.SemaphoreType.DMA,
        pltpu.SemaphoreType.DMA,
    ],
    compiler_params=pltpu.CompilerParams(needs_layout_passes=False),
)
def _onehot_sc(z_hbm, tab_hbm, out_hbm, tail_hbm,
               zbuf, zext, ztail, tabv, buf0, buf1, tbuf, sv0, sv1,
               sem0, sem1, sem2):
    wid = lax.axis_index("s") * 2 + lax.axis_index("c")
    base = jnp.minimum(wid, 4) * _COLS_HEAVY + jnp.maximum(wid - 4, 0) * _COLS_LIGHT

    pltpu.sync_copy(tab_hbm, tabv)
    pltpu.sync_copy(z_hbm.at[pl.ds(base, _COLS_LIGHT)], zbuf)

    @pl.when(wid < 4)
    def _():
        pltpu.sync_copy(z_hbm.at[pl.ds(base + _COLS_LIGHT, _CB)], zext)

    @pl.when(wid == _NW - 1)
    def _():
        pltpu.sync_copy(z_hbm.at[pl.ds(_TAIL_BASE, _TAIL_USED)], ztail)

    lanes = lax.broadcasted_iota(jnp.int32, (16,), 0)
    ones = jnp.ones((16,), jnp.float32)
    zeros = jnp.zeros((16,), jnp.float32)

    def zero_buf(buf, ngroups):
        def row(r, _):
            for g in range(ngroups):
                buf[r, pl.ds(16 * g, 16)] = zeros
            return 0
        lax.fori_loop(0, _C, row, 0)

    def out_dst(c):
        return out_hbm.at[:, pl.ds(base + c * _CB, _CB)]

    def set_chunk(zsrc, zoff, buf, sv, ngroups):
        for g in range(ngroups):
            z = zsrc[pl.ds(zoff + 16 * g, 16)]
            idx = plsc.load_gather(tabv, [z])
            cols = lanes + 16 * g
            plsc.store_scatter(buf, [idx, cols], ones)
            if sv is not None:
                sv[pl.ds(16 * g, 16)] = idx

    def clear_chunk(buf, sv):
        for g in range(_GROUPS):
            idx = sv[pl.ds(16 * g, 16)]
            cols = lanes + 16 * g
            plsc.store_scatter(buf, [idx, cols], zeros)

    zero_buf(buf0, _GROUPS)
    set_chunk(zbuf, 0, buf0, sv0, _GROUPS)
    pltpu.async_copy(buf0, out_dst(0), sem0)
    zero_buf(buf1, _GROUPS)
    set_chunk(zbuf, _CB, buf1, sv1, _GROUPS)
    pltpu.async_copy(buf1, out_dst(1), sem1)

    def pair(k, _):
        c0 = 2 * k
        pltpu.make_async_copy(buf0, out_dst(c0 - 2), sem0).wait()
        clear_chunk(buf0, sv0)
        set_chunk(zbuf, c0 * _CB, buf0, sv0, _GROUPS)
        pltpu.async_copy(buf0, out_dst(c0), sem0)
        c1 = c0 + 1
        pltpu.make_async_copy(buf1, out_dst(c1 - 2), sem1).wait()
        clear_chunk(buf1, sv1)
        set_chunk(zbuf, c1 * _CB, buf1, sv1, _GROUPS)
        pltpu.async_copy(buf1, out_dst(c1), sem1)
        return 0

    lax.fori_loop(1, 4, pair, 0)

    @pl.when(wid < 4)
    def _():
        pltpu.make_async_copy(buf0, out_dst(6), sem0).wait()
        clear_chunk(buf0, sv0)
        set_chunk(zext, 0, buf0, sv0, _GROUPS)
        pltpu.async_copy(buf0, out_dst(8), sem0)
        pltpu.make_async_copy(buf1, out_dst(7), sem1).wait()
        pltpu.make_async_copy(buf0, out_dst(8), sem0).wait()

    @pl.when(wid == _NW - 1)
    def _():
        zero_buf(tbuf, _TAIL_GROUPS)
        set_chunk(ztail, 0, tbuf, None, _TAIL_SET_GROUPS)
        pltpu.async_copy(tbuf, tail_hbm, sem2)
        pltpu.make_async_copy(buf0, out_dst(6), sem0).wait()
        pltpu.make_async_copy(buf1, out_dst(7), sem1).wait()
        pltpu.make_async_copy(tbuf, tail_hbm, sem2).wait()

    @pl.when(jnp.logical_and(wid >= 4, wid < _NW - 1))
    def _():
        pltpu.make_async_copy(buf0, out_dst(6), sem0).wait()
        pltpu.make_async_copy(buf1, out_dst(7), sem1).wait()


def kernel(Z, z_to_index):
    main, tail = _onehot_sc(Z, z_to_index)
    out = main.T
    upd = tail.T[:_TAIL_USED]
    return lax.dynamic_update_slice(out, upd, (_TAIL_BASE, 0))


# speedup vs baseline: 4.9207x; 1.0791x over previous
import functools

import jax
import jax.numpy as jnp
from jax import lax
from jax.experimental import pallas as pl
from jax.experimental.pallas import tpu as pltpu
from jax.experimental.pallas import tpu_sc as plsc

_N = 100000
_C = 100
_NW = 32
_CB = 384
_GROUPS = _CB // 16
_COLS_HEAVY = 9 * _CB
_COLS_LIGHT = 8 * _CB
_TAIL = 256
_TAIL_GROUPS = _TAIL // 16
_TAIL_BASE = 260 * _CB
_TAIL_USED = _N - _TAIL_BASE
_TAIL_SET_GROUPS = _TAIL_USED // 16

_mesh = plsc.VectorSubcoreMesh(core_axis_name="c", subcore_axis_name="s")


@functools.partial(
    pl.kernel,
    out_type=(
        jax.ShapeDtypeStruct((_C, _N), jnp.float32),
        jax.ShapeDtypeStruct((_C, _TAIL), jnp.float32),
    ),
    mesh=_mesh,
    scratch_types=[
        pltpu.VMEM((_COLS_LIGHT,), jnp.int32),
        pltpu.VMEM((_CB,), jnp.int32),
        pltpu.VMEM((_TAIL_USED,), jnp.int32),
        pltpu.VMEM((120,), jnp.int32),
        pltpu.VMEM((_C, _CB), jnp.float32),
        pltpu.VMEM((_C, _CB), jnp.float32),
        pltpu.VMEM((_C, _TAIL), jnp.float32),
        pltpu.VMEM((_CB,), jnp.int32),
        pltpu.VMEM((_CB,), jnp.int32),
        pltpu.SemaphoreType.DMA,
        pltpu.SemaphoreType.DMA,
        pltpu.SemaphoreType.DMA,
    ],
    compiler_params=pltpu.CompilerParams(needs_layout_passes=False),
)
def _onehot_sc(z_hbm, tab_hbm, out_hbm, tail_hbm,
               zbuf, zext, ztail, tabv, buf0, buf1, tbuf, sv0, sv1,
               sem0, sem1, sem2):
    wid = lax.axis_index("s") * 2 + lax.axis_index("c")
    base = jnp.minimum(wid, 4) * _COLS_HEAVY + jnp.maximum(wid - 4, 0) * _COLS_LIGHT

    pltpu.sync_copy(tab_hbm, tabv)
    pltpu.sync_copy(z_hbm.at[pl.ds(base, _COLS_LIGHT)], zbuf)

    @pl.when(wid < 4)
    def _():
        pltpu.sync_copy(z_hbm.at[pl.ds(base + _COLS_LIGHT, _CB)], zext)

    @pl.when(wid == _NW - 1)
    def _():
        pltpu.sync_copy(z_hbm.at[pl.ds(_TAIL_BASE, _TAIL_USED)], ztail)

    lanes = lax.broadcasted_iota(jnp.int32, (16,), 0)
    ones = jnp.ones((16,), jnp.float32)
    zeros = jnp.zeros((16,), jnp.float32)

    def zero_buf(buf, ngroups):
        def row(r, _):
            for g in range(ngroups):
                buf[r, pl.ds(16 * g, 16)] = zeros
            return 0
        lax.fori_loop(0, _C, row, 0)

    def out_dst(c):
        return out_hbm.at[:, pl.ds(base + c * _CB, _CB)]

    def set_chunk(zsrc, zoff, buf, sv, ngroups):
        def grp(g, _):
            z = zsrc[pl.ds(zoff + 16 * g, 16)]
            idx = plsc.load_gather(tabv, [z])
            cols = lanes + 16 * g
            plsc.store_scatter(buf, [idx, cols], ones)
            if sv is not None:
                sv[pl.ds(16 * g, 16)] = idx
            return 0
        lax.fori_loop(0, ngroups, grp, 0)

    def clear_chunk(buf, sv):
        def grp(g, _):
            idx = sv[pl.ds(16 * g, 16)]
            cols = lanes + 16 * g
            plsc.store_scatter(buf, [idx, cols], zeros)
            return 0
        lax.fori_loop(0, _GROUPS, grp, 0)

    zero_buf(buf0, _GROUPS)
    set_chunk(zbuf, 0, buf0, sv0, _GROUPS)
    pltpu.async_copy(buf0, out_dst(0), sem0)
    zero_buf(buf1, _GROUPS)
    set_chunk(zbuf, _CB, buf1, sv1, _GROUPS)
    pltpu.async_copy(buf1, out_dst(1), sem1)

    def pair(k, _):
        c0 = 2 * k
        pltpu.make_async_copy(buf0, out_dst(c0 - 2), sem0).wait()
        clear_chunk(buf0, sv0)
        set_chunk(zbuf, c0 * _CB, buf0, sv0, _GROUPS)
        pltpu.async_copy(buf0, out_dst(c0), sem0)
        c1 = c0 + 1
        pltpu.make_async_copy(buf1, out_dst(c1 - 2), sem1).wait()
        clear_chunk(buf1, sv1)
        set_chunk(zbuf, c1 * _CB, buf1, sv1, _GROUPS)
        pltpu.async_copy(buf1, out_dst(c1), sem1)
        return 0

    lax.fori_loop(1, 4, pair, 0)

    @pl.when(wid < 4)
    def _():
        pltpu.make_async_copy(buf0, out_dst(6), sem0).wait()
        clear_chunk(buf0, sv0)
        set_chunk(zext, 0, buf0, sv0, _GROUPS)
        pltpu.async_copy(buf0, out_dst(8), sem0)
        pltpu.make_async_copy(buf1, out_dst(7), sem1).wait()
        pltpu.make_async_copy(buf0, out_dst(8), sem0).wait()

    @pl.when(wid == _NW - 1)
    def _():
        zero_buf(tbuf, _TAIL_GROUPS)
        set_chunk(ztail, 0, tbuf, None, _TAIL_SET_GROUPS)
        pltpu.async_copy(tbuf, tail_hbm, sem2)
        pltpu.make_async_copy(buf0, out_dst(6), sem0).wait()
        pltpu.make_async_copy(buf1, out_dst(7), sem1).wait()
        pltpu.make_async_copy(tbuf, tail_hbm, sem2).wait()

    @pl.when(jnp.logical_and(wid >= 4, wid < _NW - 1))
    def _():
        pltpu.make_async_copy(buf0, out_dst(6), sem0).wait()
        pltpu.make_async_copy(buf1, out_dst(7), sem1).wait()


def kernel(Z, z_to_index):
    main, tail = _onehot_sc(Z, z_to_index)
    out = main.T
    upd = tail.T[:_TAIL_USED]
    return lax.dynamic_update_slice(out, upd, (_TAIL_BASE, 0))
